# baseline (speedup 1.0000x reference)
import functools

import jax
import jax.numpy as jnp
from jax import lax
from jax.experimental import pallas as pl
from jax.experimental.pallas import tpu as pltpu
from jax.experimental.pallas import tpu_sc as plsc


def _make_gather(B, H, V, D, NW):
    LB = 128
    DT = D // 8
    n_units = H * (B // LB)
    n_per_w = n_units // NW
    assert n_units % NW == 0 and n_per_w >= 4
    out_rows = H * DT * (B // LB) * 8
    mesh = plsc.VectorSubcoreMesh(core_axis_name="c", subcore_axis_name="s")

    @functools.partial(
        pl.kernel,
        out_type=jax.ShapeDtypeStruct((out_rows, LB), jnp.float32),
        mesh=mesh,
        scratch_types=(
            [pltpu.VMEM((LB,), jnp.int32) for _ in range(2)]
            + [pltpu.VMEM((LB, D), jnp.float32) for _ in range(2)]
            + [pltpu.VMEM((D, LB), jnp.float32) for _ in range(2)]
            + [pltpu.SemaphoreType.DMA for _ in range(6)]
        ),
        compiler_params=pltpu.CompilerParams(use_tc_tiling_on_sc=False, needs_layout_passes=False),
    )
    def gather_kernel(ids_hbm, table_hbm, out_hbm, *scratch):
        idx_v = scratch[0:2]
        rows_v = scratch[2:4]
        tile_v = scratch[4:6]
        si = scratch[6:8]
        sg = scratch[8:10]
        so = scratch[10:12]
        wid = lax.axis_index("s") * 2 + lax.axis_index("c")
        u0 = wid * n_per_w
        lanes = lax.iota(jnp.int32, 16)

        def fire_idx(k, b):
            u = u0 + jnp.minimum(k, n_per_w - 1)
            h = u // (B // LB)
            bt = u % (B // LB)
            pltpu.async_copy(ids_hbm.at[h, pl.ds(bt * LB, LB)],
                             idx_v[b], si[b])

        def fire_gather(b):
            pltpu.async_copy(table_hbm.at[idx_v[b]], rows_v[b], sg[b])

        def fire_outs(k, b):
            u = u0 + k
            h = u // (B // LB)
            bt = u % (B // LB)
            for dt in range(DT):
                r0 = h * (DT * (B // LB) * 8) + dt * ((B // LB) * 8) + bt * 8
                pltpu.async_copy(tile_v[b].at[pl.ds(dt * 8, 8), :],
                                 out_hbm.at[pl.ds(r0, 8)], so[b])

        def wait_idx(b):
            pltpu.make_async_copy(ids_hbm.at[0, pl.ds(0, LB)],
                                  idx_v[b], si[b]).wait()

        def wait_gather(b):
            pltpu.make_async_copy(table_hbm.at[idx_v[b]],
                                  rows_v[b], sg[b]).wait()

        def wait_outs(b):
            for dt in range(DT):
                pltpu.make_async_copy(tile_v[b].at[pl.ds(dt * 8, 8), :],
                                      out_hbm.at[pl.ds(0, 8)], so[b]).wait()

        def transpose(b):
            rows = rows_v[b]
            tile = tile_v[b]

            @pl.loop(0, D)
            def _(d):
                col = jnp.full((16,), d, jnp.int32)
                for g in range(LB // 16):
                    v = plsc.load_gather(rows, [g * 16 + lanes, col])
                    tile[d, pl.ds(g * 16, 16)] = v

        fire_idx(0, 0)
        fire_idx(1, 1)
        wait_idx(0)
        fire_gather(0)

        def step(k, b, first, last):
            wait_gather(b)
            if not last:
                wait_idx(1 - b)
                fire_gather(1 - b)
            fire_idx(k + 2, b)
            if not first:
                wait_outs(b)
            transpose(b)
            fire_outs(k, b)

        step(0, 0, True, False)
        step(1, 1, True, False)

        @pl.loop(1, n_per_w // 2 - 1)
        def _(p):
            for b in range(2):
                step(2 * p + b, b, False, False)

        step(n_per_w - 2, 0, False, False)
        step(n_per_w - 1, 1, False, True)

        for b in range(2):
            wait_idx(b)
            wait_outs(b)

    return gather_kernel


def kernel(word_ids, table):
    B, H = word_ids.shape
    V, D = table.shape
    NW = 32
    ids_t = word_ids.T.astype(jnp.int32)
    out2d = _make_gather(B, H, V, D, NW)(ids_t, table)
    out5 = out2d.reshape(H, D // 8, B // 128, 8, 128)
    return out5.transpose(2, 4, 0, 1, 3).reshape(B, H, D)


# speedup vs baseline: 1.7871x; 1.7871x over previous
import functools

import jax
import jax.numpy as jnp
from jax import lax
from jax.experimental import pallas as pl
from jax.experimental.pallas import tpu as pltpu
from jax.experimental.pallas import tpu_sc as plsc


def _make_gather(B, H, V, D, NW):
    LB = 128
    DT = D // 8
    n_units = H * (B // LB)
    n_per_w = n_units // NW
    assert n_units % NW == 0 and n_per_w >= 4
    out_rows = H * DT * (B // LB) * 8
    mesh = plsc.VectorSubcoreMesh(core_axis_name="c", subcore_axis_name="s")

    @functools.partial(
        pl.kernel,
        out_type=jax.ShapeDtypeStruct((out_rows, LB), jnp.float32),
        mesh=mesh,
        scratch_types=(
            [pltpu.VMEM((LB,), jnp.int32) for _ in range(2)]
            + [pltpu.VMEM((LB, D), jnp.float32) for _ in range(2)]
            + [pltpu.VMEM((D, LB), jnp.float32) for _ in range(2)]
            + [pltpu.SemaphoreType.DMA for _ in range(6)]
        ),
        compiler_params=pltpu.CompilerParams(use_tc_tiling_on_sc=False, needs_layout_passes=False),
    )
    def gather_kernel(ids_hbm, table_hbm, out_hbm, *scratch):
        idx_v = scratch[0:2]
        rows_v = scratch[2:4]
        tile_v = scratch[4:6]
        si = scratch[6:8]
        sg = scratch[8:10]
        so = scratch[10:12]
        wid = lax.axis_index("s") * 2 + lax.axis_index("c")
        u0 = wid * n_per_w
        lanes = lax.iota(jnp.int32, 16)

        def fire_idx(k, b):
            u = u0 + jnp.minimum(k, n_per_w - 1)
            h = u // (B // LB)
            bt = u % (B // LB)
            pltpu.async_copy(ids_hbm.at[h, pl.ds(bt * LB, LB)],
                             idx_v[b], si[b])

        def fire_gather(b):
            pltpu.async_copy(table_hbm.at[idx_v[b]], rows_v[b], sg[b])

        def fire_outs(k, b):
            u = u0 + k
            h = u // (B // LB)
            bt = u % (B // LB)
            for dt in range(DT):
                r0 = h * (DT * (B // LB) * 8) + dt * ((B // LB) * 8) + bt * 8
                pltpu.async_copy(tile_v[b].at[pl.ds(dt * 8, 8), :],
                                 out_hbm.at[pl.ds(r0, 8)], so[b])

        def wait_idx(b):
            pltpu.make_async_copy(ids_hbm.at[0, pl.ds(0, LB)],
                                  idx_v[b], si[b]).wait()

        def wait_gather(b):
            pltpu.make_async_copy(table_hbm.at[idx_v[b]],
                                  rows_v[b], sg[b]).wait()

        def wait_outs(b):
            for dt in range(DT):
                pltpu.make_async_copy(tile_v[b].at[pl.ds(dt * 8, 8), :],
                                      out_hbm.at[pl.ds(0, 8)], so[b]).wait()

        def transpose(b):
            rows = rows_v[b]
            tile = tile_v[b]

            @plsc.parallel_loop(0, D, unroll=4)
            def _(d):
                col = jnp.full((16,), d, jnp.int32)
                for g in range(LB // 16):
                    v = plsc.load_gather(rows, [g * 16 + lanes, col])
                    tile[d, pl.ds(g * 16, 16)] = v

        fire_idx(0, 0)
        fire_idx(1, 1)
        wait_idx(0)
        fire_gather(0)

        def step(k, b, first, last):
            wait_gather(b)
            if not last:
                wait_idx(1 - b)
                fire_gather(1 - b)
            fire_idx(k + 2, b)
            if not first:
                wait_outs(b)
            transpose(b)
            fire_outs(k, b)

        step(0, 0, True, False)
        step(1, 1, True, False)

        @pl.loop(1, n_per_w // 2 - 1)
        def _(p):
            for b in range(2):
                step(2 * p + b, b, False, False)

        step(n_per_w - 2, 0, False, False)
        step(n_per_w - 1, 1, False, True)

        for b in range(2):
            wait_idx(b)
            wait_outs(b)

    return gather_kernel


def kernel(word_ids, table):
    B, H = word_ids.shape
    V, D = table.shape
    NW = 32
    ids_t = word_ids.T.astype(jnp.int32)
    out2d = _make_gather(B, H, V, D, NW)(ids_t, table)
    out5 = out2d.reshape(H, D // 8, B // 128, 8, 128)
    return out5.transpose(2, 4, 0, 1, 3).reshape(B, H, D)


# speedup vs baseline: 4.1461x; 2.3201x over previous
import functools

import jax
import jax.numpy as jnp
from jax import lax
from jax.experimental import pallas as pl
from jax.experimental.pallas import tpu as pltpu
from jax.experimental.pallas import tpu_sc as plsc


def _make_gather(B, H, V, D, NW):
    LB = 128
    DT = D // 8
    n_units = H * (B // LB)
    n_per_w = n_units // NW
    assert n_units % NW == 0 and n_per_w >= 4
    out_rows = H * DT * (B // LB) * 8
    mesh = plsc.VectorSubcoreMesh(core_axis_name="c", subcore_axis_name="s")

    @functools.partial(
        pl.kernel,
        out_type=jax.ShapeDtypeStruct((out_rows, LB), jnp.float32),
        mesh=mesh,
        scratch_types=(
            [pltpu.VMEM((LB,), jnp.int32) for _ in range(2)]
            + [pltpu.VMEM((LB, D), jnp.float32) for _ in range(2)]
            + [pltpu.VMEM((D, LB + 1), jnp.float32) for _ in range(2)]
            + [pltpu.SemaphoreType.DMA for _ in range(6)]
        ),
        compiler_params=pltpu.CompilerParams(use_tc_tiling_on_sc=False, needs_layout_passes=False),
    )
    def gather_kernel(ids_hbm, table_hbm, out_hbm, *scratch):
        idx_v = scratch[0:2]
        rows_v = scratch[2:4]
        tile_v = scratch[4:6]
        si = scratch[6:8]
        sg = scratch[8:10]
        so = scratch[10:12]
        wid = lax.axis_index("s") * 2 + lax.axis_index("c")
        u0 = wid * n_per_w
        lanes = lax.iota(jnp.int32, 16)

        def fire_idx(k, b):
            u = u0 + jnp.minimum(k, n_per_w - 1)
            h = u // (B // LB)
            bt = u % (B // LB)
            pltpu.async_copy(ids_hbm.at[h, pl.ds(bt * LB, LB)],
                             idx_v[b], si[b])

        def fire_gather(b):
            pltpu.async_copy(table_hbm.at[idx_v[b]], rows_v[b], sg[b])

        def fire_outs(k, b):
            u = u0 + k
            h = u // (B // LB)
            bt = u % (B // LB)
            for dt in range(DT):
                r0 = h * (DT * (B // LB) * 8) + dt * ((B // LB) * 8) + bt * 8
                pltpu.async_copy(tile_v[b].at[pl.ds(dt * 8, 8), pl.ds(0, LB)],
                                 out_hbm.at[pl.ds(r0, 8)], so[b])

        def wait_idx(b):
            pltpu.make_async_copy(ids_hbm.at[0, pl.ds(0, LB)],
                                  idx_v[b], si[b]).wait()

        def wait_gather(b):
            pltpu.make_async_copy(table_hbm.at[idx_v[b]],
                                  rows_v[b], sg[b]).wait()

        def wait_outs(b):
            for dt in range(DT):
                pltpu.make_async_copy(tile_v[b].at[pl.ds(dt * 8, 8), pl.ds(0, LB)],
                                      out_hbm.at[pl.ds(0, 8)], so[b]).wait()

        def transpose(b):
            rows = rows_v[b]
            tile = tile_v[b]

            @plsc.parallel_loop(0, LB, unroll=4)
            def _(bi):
                col = jnp.full((16,), bi, jnp.int32)
                for gd in range(D // 16):
                    v = rows[bi, pl.ds(gd * 16, 16)]
                    plsc.store_scatter(tile, [gd * 16 + lanes, col], v)

        fire_idx(0, 0)
        fire_idx(1, 1)
        wait_idx(0)
        fire_gather(0)

        def step(k, b, first, last):
            wait_gather(b)
            if not last:
                wait_idx(1 - b)
                fire_gather(1 - b)
            fire_idx(k + 2, b)
            if not first:
                wait_outs(b)
            transpose(b)
            fire_outs(k, b)

        step(0, 0, True, False)
        step(1, 1, True, False)

        @pl.loop(1, n_per_w // 2 - 1)
        def _(p):
            for b in range(2):
                step(2 * p + b, b, False, False)

        step(n_per_w - 2, 0, False, False)
        step(n_per_w - 1, 1, False, True)

        for b in range(2):
            wait_idx(b)
            wait_outs(b)

    return gather_kernel


def kernel(word_ids, table):
    B, H = word_ids.shape
    V, D = table.shape
    NW = 32
    ids_t = word_ids.T.astype(jnp.int32)
    out2d = _make_gather(B, H, V, D, NW)(ids_t, table)
    out5 = out2d.reshape(H, D // 8, B // 128, 8, 128)
    return out5.transpose(2, 4, 0, 1, 3).reshape(B, H, D)


# speedup vs baseline: 5.2450x; 1.2650x over previous
import functools

import jax
import jax.numpy as jnp
from jax import lax
from jax.experimental import pallas as pl
from jax.experimental.pallas import tpu as pltpu
from jax.experimental.pallas import tpu_sc as plsc


def _make_gather(B, H, V, D, NW):
    LB = 256
    NT = LB // 128
    DT = D // 8
    n_units = H * (B // LB)
    n_per_w = n_units // NW
    assert n_units % NW == 0 and n_per_w >= 4
    out_rows = H * DT * (B // 128) * 8
    mesh = plsc.VectorSubcoreMesh(core_axis_name="c", subcore_axis_name="s")

    @functools.partial(
        pl.kernel,
        out_type=jax.ShapeDtypeStruct((out_rows, 128), jnp.float32),
        mesh=mesh,
        scratch_types=(
            [pltpu.VMEM((LB,), jnp.int32) for _ in range(2)]
            + [pltpu.VMEM((LB, D), jnp.float32) for _ in range(2)]
            + [pltpu.VMEM((D, LB + 1), jnp.float32) for _ in range(2)]
            + [pltpu.SemaphoreType.DMA for _ in range(6)]
        ),
        compiler_params=pltpu.CompilerParams(use_tc_tiling_on_sc=False, needs_layout_passes=False),
    )
    def gather_kernel(ids_hbm, table_hbm, out_hbm, *scratch):
        idx_v = scratch[0:2]
        rows_v = scratch[2:4]
        tile_v = scratch[4:6]
        si = scratch[6:8]
        sg = scratch[8:10]
        so = scratch[10:12]
        wid = lax.axis_index("s") * 2 + lax.axis_index("c")
        u0 = wid * n_per_w
        lanes = lax.iota(jnp.int32, 16)

        def fire_idx(k, b):
            u = u0 + jnp.minimum(k, n_per_w - 1)
            h = u // (B // LB)
            bt = u % (B // LB)
            pltpu.async_copy(ids_hbm.at[h, pl.ds(bt * LB, LB)],
                             idx_v[b], si[b])

        def fire_gather(b):
            pltpu.async_copy(table_hbm.at[idx_v[b]], rows_v[b], sg[b])

        def fire_outs(k, b):
            u = u0 + k
            h = u // (B // LB)
            bq = u % (B // LB)
            for dt in range(DT):
                for t in range(NT):
                    r0 = (h * (DT * (B // 128) * 8) + dt * ((B // 128) * 8)
                          + (bq * NT + t) * 8)
                    pltpu.async_copy(
                        tile_v[b].at[pl.ds(dt * 8, 8), pl.ds(t * 128, 128)],
                        out_hbm.at[pl.ds(r0, 8)], so[b])

        def wait_idx(b):
            pltpu.make_async_copy(ids_hbm.at[0, pl.ds(0, LB)],
                                  idx_v[b], si[b]).wait()

        def wait_gather(b):
            pltpu.make_async_copy(table_hbm.at[idx_v[b]],
                                  rows_v[b], sg[b]).wait()

        def wait_outs(b):
            for dt in range(DT):
                for t in range(NT):
                    pltpu.make_async_copy(
                        tile_v[b].at[pl.ds(dt * 8, 8), pl.ds(t * 128, 128)],
                        out_hbm.at[pl.ds(0, 8)], so[b]).wait()

        def transpose(b):
            rows = rows_v[b]
            tile = tile_v[b]

            @plsc.parallel_loop(0, LB, unroll=4)
            def _(bi):
                col = jnp.full((16,), bi, jnp.int32)
                for gd in range(D // 16):
                    v = rows[bi, pl.ds(gd * 16, 16)]
                    plsc.store_scatter(tile, [gd * 16 + lanes, col], v)

        fire_idx(0, 0)
        fire_idx(1, 1)
        wait_idx(0)
        fire_gather(0)

        def step(k, b, first, last):
            wait_gather(b)
            if not last:
                wait_idx(1 - b)
                fire_gather(1 - b)
            fire_idx(k + 2, b)
            if not first:
                wait_outs(b)
            transpose(b)
            fire_outs(k, b)

        step(0, 0, True, False)
        step(1, 1, True, False)

        @pl.loop(1, n_per_w // 2 - 1)
        def _(p):
            for b in range(2):
                step(2 * p + b, b, False, False)

        step(n_per_w - 2, 0, False, False)
        step(n_per_w - 1, 1, False, True)

        for b in range(2):
            wait_idx(b)
            wait_outs(b)

    return gather_kernel


def kernel(word_ids, table):
    B, H = word_ids.shape
    V, D = table.shape
    NW = 32
    ids_t = word_ids.T.astype(jnp.int32)
    out2d = _make_gather(B, H, V, D, NW)(ids_t, table)
    out5 = out2d.reshape(H, D // 8, B // 128, 8, 128)
    return out5.transpose(2, 4, 0, 1, 3).reshape(B, H, D)


# speedup vs baseline: 5.5410x; 1.0564x over previous
import functools

import jax
import jax.numpy as jnp
from jax import lax
from jax.experimental import pallas as pl
from jax.experimental.pallas import tpu as pltpu
from jax.experimental.pallas import tpu_sc as plsc


def _make_gather(B, H, V, D, NW):
    LB = 256
    NT = LB // 128
    DT = D // 8
    n_units = H * (B // LB)
    n_per_w = n_units // NW
    assert n_units % NW == 0 and n_per_w >= 12
    out_rows = H * DT * (B // 128) * 8
    n_steady = (n_per_w - 6) // 6
    tail_start = 6 + 6 * n_steady
    mesh = plsc.VectorSubcoreMesh(core_axis_name="c", subcore_axis_name="s")

    @functools.partial(
        pl.kernel,
        out_type=jax.ShapeDtypeStruct((out_rows, 128), jnp.float32),
        mesh=mesh,
        scratch_types=(
            [pltpu.VMEM((LB,), jnp.int32) for _ in range(3)]
            + [pltpu.VMEM((LB, D), jnp.float32) for _ in range(3)]
            + [pltpu.VMEM((D, LB + 1), jnp.float32) for _ in range(2)]
            + [pltpu.SemaphoreType.DMA for _ in range(8)]
        ),
        compiler_params=pltpu.CompilerParams(
            use_tc_tiling_on_sc=False, needs_layout_passes=False),
    )
    def gather_kernel(ids_hbm, table_hbm, out_hbm, *scratch):
        idx_v = scratch[0:3]
        rows_v = scratch[3:6]
        tile_v = scratch[6:8]
        si = scratch[8:11]
        sg = scratch[11:14]
        so = scratch[14:16]
        wid = lax.axis_index("s") * 2 + lax.axis_index("c")
        u0 = wid * n_per_w
        lanes = lax.iota(jnp.int32, 16)

        def fire_idx(k, b):
            u = u0 + jnp.minimum(k, n_per_w - 1)
            h = u // (B // LB)
            bq = u % (B // LB)
            pltpu.async_copy(ids_hbm.at[h, pl.ds(bq * LB, LB)],
                             idx_v[b], si[b])

        def fire_gather(b):
            pltpu.async_copy(table_hbm.at[idx_v[b]], rows_v[b], sg[b])

        def fire_outs(k, b):
            u = u0 + k
            h = u // (B // LB)
            bq = u % (B // LB)
            for dt in range(DT):
                for t in range(NT):
                    r0 = (h * (DT * (B // 128) * 8) + dt * ((B // 128) * 8)
                          + (bq * NT + t) * 8)
                    pltpu.async_copy(
                        tile_v[b].at[pl.ds(dt * 8, 8), pl.ds(t * 128, 128)],
                        out_hbm.at[pl.ds(r0, 8)], so[b])

        def wait_idx(b):
            pltpu.make_async_copy(ids_hbm.at[0, pl.ds(0, LB)],
                                  idx_v[b], si[b]).wait()

        def wait_gather(b):
            pltpu.make_async_copy(table_hbm.at[idx_v[b]],
                                  rows_v[b], sg[b]).wait()

        def wait_outs(b):
            for dt in range(DT):
                for t in range(NT):
                    pltpu.make_async_copy(
                        tile_v[b].at[pl.ds(dt * 8, 8), pl.ds(t * 128, 128)],
                        out_hbm.at[pl.ds(0, 8)], so[b]).wait()

        def transpose(b3, b2):
            rows = rows_v[b3]
            tile = tile_v[b2]

            @plsc.parallel_loop(0, LB, unroll=4)
            def _(bi):
                col = jnp.full((16,), bi, jnp.int32)
                for gd in range(D // 16):
                    v = rows[bi, pl.ds(gd * 16, 16)]
                    plsc.store_scatter(tile, [gd * 16 + lanes, col], v)

        def step(k, b3, b2, first, fire_g):
            wait_gather(b3)
            if fire_g:
                b_next = (b3 + 2) % 3
                wait_idx(b_next)
                fire_gather(b_next)
            fire_idx(k + 3, b3)
            if not first:
                wait_outs(b2)
            transpose(b3, b2)
            fire_outs(k, b2)

        for j in range(3):
            fire_idx(j, j)
        wait_idx(0)
        fire_gather(0)
        wait_idx(1)
        fire_gather(1)

        for k in range(6):
            step(k, k % 3, k % 2, k < 2, True)

        @pl.loop(1, n_steady + 1)
        def _(g):
            for j in range(6):
                k = 6 * g + j
                step(k, j % 3, j % 2, False, True)

        for k in range(tail_start, n_per_w):
            step(k, k % 3, k % 2, False, k + 2 < n_per_w)

        for b in range(3):
            wait_idx(b)
        for b in range(2):
            wait_outs(b)

    return gather_kernel


def kernel(word_ids, table):
    B, H = word_ids.shape
    V, D = table.shape
    NW = 32
    ids_t = word_ids.T.astype(jnp.int32)
    out2d = _make_gather(B, H, V, D, NW)(ids_t, table)
    out5 = out2d.reshape(H, D // 8, B // 128, 8, 128)
    return out5.transpose(2, 4, 0, 1, 3).reshape(B, H, D)
